# Initial kernel scaffold; baseline (speedup 1.0000x reference)
import jax
import jax.numpy as jnp
from jax.experimental import pallas as pl


def kernel(x, edge_index, W1, b1, W2, b2):
    raise NotImplementedError("write your pallas kernel here")


# speedup vs baseline: 12.5646x; 12.5646x over previous
import functools

import jax
import jax.numpy as jnp
from jax import lax
from jax.experimental import pallas as pl
from jax.experimental.pallas import tpu as pltpu
from jax.experimental.pallas import tpu_sc as plsc

_NC = 2
_NS = 16


@functools.lru_cache(maxsize=None)
def _make_deg_kernel(n, e):
    nw = _NC * _NS
    e_per_tile = e // nw
    k = 80
    assert e % nw == 0 and e_per_tile % k == 0
    n_chunks = e_per_tile // k
    rows_per_sub = (pl.cdiv(n, _NS) + 7) // 8 * 8
    n_pad = rows_per_sub * _NS
    w = 16

    mesh = plsc.VectorSubcoreMesh(core_axis_name="c", subcore_axis_name="s")

    @functools.partial(
        pl.kernel,
        out_type=jax.ShapeDtypeStruct((_NC, n_pad, w), jnp.float32),
        mesh=mesh,
        scratch_types=[
            pltpu.VMEM((k,), jnp.int32),
            pltpu.VMEM((k, w), jnp.float32),
            pltpu.VMEM_SHARED((n_pad, w), jnp.float32),
        ],
    )
    def deg_kernel(dst_hbm, zeros_hbm, out_hbm, dst_v, ones_v, acc):
        c = lax.axis_index("c")
        s = lax.axis_index("s")
        r0 = s * rows_per_sub
        pltpu.sync_copy(zeros_hbm.at[pl.ds(r0, rows_per_sub)],
                        acc.at[pl.ds(r0, rows_per_sub)])
        for i in range(k):
            ones_v[i, :] = jnp.full((w,), 1.0, jnp.float32)
        plsc.subcore_barrier()

        wid = s * _NC + c
        base = wid * e_per_tile

        def body(i, carry):
            pltpu.sync_copy(dst_hbm.at[pl.ds(base + i * k, k)], dst_v)
            pltpu.sync_copy(ones_v, acc.at[dst_v], add=True)
            return carry

        lax.fori_loop(0, n_chunks, body, 0)
        plsc.subcore_barrier()
        pltpu.sync_copy(acc.at[pl.ds(r0, rows_per_sub)],
                        out_hbm.at[c, pl.ds(r0, rows_per_sub)])

    return deg_kernel


@functools.lru_cache(maxsize=None)
def _make_agg_kernel(n, e, d):
    nw = _NC * _NS
    e_per_tile = e // nw
    k = 80
    assert e % nw == 0 and e_per_tile % k == 0
    n_chunks = e_per_tile // k
    rows_per_sub = (pl.cdiv(n, _NS) + 7) // 8 * 8
    n_pad = rows_per_sub * _NS

    mesh = plsc.VectorSubcoreMesh(core_axis_name="c", subcore_axis_name="s")

    @functools.partial(
        pl.kernel,
        out_type=jax.ShapeDtypeStruct((_NC, n_pad, d), jnp.float32),
        mesh=mesh,
        scratch_types=[
            pltpu.VMEM((k,), jnp.int32),
            pltpu.VMEM((k,), jnp.int32),
            pltpu.VMEM((k, d), jnp.float32),
            pltpu.VMEM_SHARED((n_pad, d), jnp.float32),
            pltpu.SemaphoreType.DMA,
        ],
    )
    def agg_kernel(hp_hbm, src_hbm, dst_hbm, zeros_hbm, out_hbm,
                   src_v, dst_v, rows_v, acc, sem):
        c = lax.axis_index("c")
        s = lax.axis_index("s")
        r0 = s * rows_per_sub
        pltpu.sync_copy(zeros_hbm.at[pl.ds(r0, rows_per_sub)],
                        acc.at[pl.ds(r0, rows_per_sub)])
        plsc.subcore_barrier()

        wid = s * _NC + c
        base = wid * e_per_tile

        def body(i, carry):
            off = base + i * k
            pltpu.sync_copy(src_hbm.at[pl.ds(off, k)], src_v)
            pltpu.sync_copy(dst_hbm.at[pl.ds(off, k)], dst_v)
            pltpu.async_copy(hp_hbm.at[src_v], rows_v, sem).wait()
            pltpu.sync_copy(rows_v, acc.at[dst_v], add=True)
            return carry

        lax.fori_loop(0, n_chunks, body, 0)
        plsc.subcore_barrier()
        pltpu.sync_copy(acc.at[pl.ds(r0, rows_per_sub)],
                        out_hbm.at[c, pl.ds(r0, rows_per_sub)])

    return agg_kernel


_BN = 1000


def _prep_body(pdeg_ref, x_ref, w_ref, hp_ref, dinv_ref):
    deg = pdeg_ref[0, :, 0:1] + pdeg_ref[1, :, 0:1] + 1.0
    dinv = lax.rsqrt(deg)
    u = jnp.dot(x_ref[...], w_ref[...], preferred_element_type=jnp.float32)
    hp_ref[...] = u * dinv
    dinv_ref[...] = dinv


def _mid_body(pa_ref, hp_ref, dinv_ref, b_ref, w_ref, out_ref):
    s = pa_ref[0] + pa_ref[1] + hp_ref[...]
    h1 = s * dinv_ref[...] + b_ref[...]
    x2 = jnp.maximum(h1, 0.0)
    u = jnp.dot(x2, w_ref[...], preferred_element_type=jnp.float32)
    out_ref[...] = u * dinv_ref[...]


def _final_body(pa_ref, hp_ref, dinv_ref, b_ref, out_ref):
    s = pa_ref[0] + pa_ref[1] + hp_ref[...]
    out_ref[...] = s * dinv_ref[...] + b_ref[...]


def _tc_prep(pdeg, x, w1):
    n, d_in = x.shape
    d_hid = w1.shape[1]
    grid = n // _BN
    return pl.pallas_call(
        _prep_body,
        grid=(grid,),
        in_specs=[
            pl.BlockSpec((_NC, _BN, pdeg.shape[2]), lambda i: (0, i, 0)),
            pl.BlockSpec((_BN, d_in), lambda i: (i, 0)),
            pl.BlockSpec((d_in, d_hid), lambda i: (0, 0)),
        ],
        out_specs=[
            pl.BlockSpec((_BN, d_hid), lambda i: (i, 0)),
            pl.BlockSpec((_BN, 1), lambda i: (i, 0)),
        ],
        out_shape=[
            jax.ShapeDtypeStruct((n, d_hid), jnp.float32),
            jax.ShapeDtypeStruct((n, 1), jnp.float32),
        ],
    )(pdeg, x, w1)


def _tc_mid(pagg, hp, dinv, b1, w2):
    n, d = hp.shape
    d_out = w2.shape[1]
    grid = n // _BN
    return pl.pallas_call(
        _mid_body,
        grid=(grid,),
        in_specs=[
            pl.BlockSpec((_NC, _BN, d), lambda i: (0, i, 0)),
            pl.BlockSpec((_BN, d), lambda i: (i, 0)),
            pl.BlockSpec((_BN, 1), lambda i: (i, 0)),
            pl.BlockSpec((1, d), lambda i: (0, 0)),
            pl.BlockSpec((d, d_out), lambda i: (0, 0)),
        ],
        out_specs=pl.BlockSpec((_BN, d_out), lambda i: (i, 0)),
        out_shape=jax.ShapeDtypeStruct((n, d_out), jnp.float32),
    )(pagg, hp, dinv, b1.reshape(1, d), w2)


def _tc_final(pagg, hp, dinv, b2):
    n, d = hp.shape
    grid = n // _BN
    return pl.pallas_call(
        _final_body,
        grid=(grid,),
        in_specs=[
            pl.BlockSpec((_NC, _BN, d), lambda i: (0, i, 0)),
            pl.BlockSpec((_BN, d), lambda i: (i, 0)),
            pl.BlockSpec((_BN, 1), lambda i: (i, 0)),
            pl.BlockSpec((1, d), lambda i: (0, 0)),
        ],
        out_specs=pl.BlockSpec((_BN, d), lambda i: (i, 0)),
        out_shape=jax.ShapeDtypeStruct((n, d), jnp.float32),
    )(pagg, hp, dinv, b2.reshape(1, d))


def kernel(x, edge_index, W1, b1, W2, b2):
    n, d = x.shape
    e = edge_index.shape[1]
    src = edge_index[0]
    dst = edge_index[1]

    n_pad = ((pl.cdiv(n, _NS) + 7) // 8 * 8) * _NS
    zeros_w = jnp.zeros((n_pad, 16), jnp.float32)
    zeros_d = jnp.zeros((n_pad, d), jnp.float32)

    pdeg = _make_deg_kernel(n, e)(dst, zeros_w)[:, :n]
    agg = _make_agg_kernel(n, e, d)

    hp1, dinv = _tc_prep(pdeg, x, W1)
    p1 = agg(hp1, src, dst, zeros_d)[:, :n]
    hp2 = _tc_mid(p1, hp1, dinv, b1, W2)
    p2 = agg(hp2, src, dst, zeros_d)[:, :n]
    return _tc_final(p2, hp2, dinv, b2)
